# Initial kernel scaffold; baseline (speedup 1.0000x reference)
import jax
import jax.numpy as jnp
from jax.experimental import pallas as pl


def kernel(x, u, v, adj, W1, b1, W2, b2, Wq, Wk, fc_W, fc_b):
    raise NotImplementedError("write your pallas kernel here")


# speedup vs baseline: 175.9871x; 175.9871x over previous
import functools

import jax
import jax.numpy as jnp
from jax import lax
from jax.experimental import pallas as pl
from jax.experimental.pallas import tpu as pltpu
from jax.experimental.pallas import tpu_sc as plsc

N = 10000
D_IN = 128
D_H = 256
D_OUT = 128
P = 3
L = 4
B = 1024

NPAD = 10240
TM = 400
TK = 2048
BIGI = 2 ** 30


def _xw_body(x_ref, w_ref, o_ref):
    o_ref[...] = jnp.dot(x_ref[...], w_ref[...],
                         preferred_element_type=jnp.float32)


def _small_matmul(x, w, tm):
    m, k = x.shape
    n = w.shape[1]
    return pl.pallas_call(
        _xw_body,
        grid=(m // tm,),
        in_specs=[pl.BlockSpec((tm, k), lambda i: (i, 0)),
                  pl.BlockSpec((k, n), lambda i: (0, 0))],
        out_specs=pl.BlockSpec((tm, n), lambda i: (i, 0)),
        out_shape=jax.ShapeDtypeStruct((m, n), jnp.float32),
    )(x, w)


def _top3_tile(a, iota):
    vals, idxs = [], []
    for _ in range(3):
        m = jnp.max(a, axis=1, keepdims=True)
        ii = jnp.min(jnp.where(a == m, iota, BIGI), axis=1, keepdims=True)
        vals.append(m)
        idxs.append(ii)
        a = jnp.where(iota == ii, -1.0, a)
    return vals, idxs


def _merge3(cv, ci):
    nv, ni = [], []
    for _ in range(3):
        m = jnp.max(cv, axis=1, keepdims=True)
        ii = jnp.min(jnp.where(cv == m, ci, BIGI), axis=1, keepdims=True)
        nv.append(m)
        ni.append(ii)
        cv = jnp.where(ci == ii, -1.0, cv)
    return nv, ni


def _mm1_body(adj_ref, xw1_ref, b1_ref, h1_ref, top3_ref,
              acc_ref, tv_ref, ti_ref):
    k = pl.program_id(1)
    nk = pl.num_programs(1)

    @pl.when(k == 0)
    def _():
        acc_ref[...] = jnp.zeros_like(acc_ref)
        tv_ref[...] = jnp.full_like(tv_ref, -1.0)
        ti_ref[...] = jnp.full_like(ti_ref, BIGI)

    a = adj_ref[...]
    acc_ref[...] += jnp.dot(a, xw1_ref[...],
                            preferred_element_type=jnp.float32)

    iota = (lax.broadcasted_iota(jnp.int32, (TM, TK), 1) + k * TK)
    vals, idxs = _top3_tile(a, iota)
    cv = jnp.concatenate([tv_ref[:, :3]] + vals, axis=1)
    ci = jnp.concatenate([ti_ref[:, :3]] + idxs, axis=1)
    nv, ni = _merge3(cv, ci)
    tv_ref[:, :3] = jnp.concatenate(nv, axis=1)
    ti_ref[:, :3] = jnp.concatenate(ni, axis=1)

    @pl.when(k == nk - 1)
    def _():
        h1_ref[...] = jnp.maximum(acc_ref[...] + b1_ref[...], 0.0)
        top3_ref[...] = ti_ref[...]


def _mm2_body(adj_ref, hw_ref, b2_ref, emb_ref, acc_ref):
    k = pl.program_id(1)
    nk = pl.num_programs(1)

    @pl.when(k == 0)
    def _():
        acc_ref[...] = jnp.zeros_like(acc_ref)

    acc_ref[...] += jnp.dot(adj_ref[...], hw_ref[...],
                            preferred_element_type=jnp.float32)

    @pl.when(k == nk - 1)
    def _():
        emb_ref[...] = acc_ref[...] + b2_ref[...]


def _gcn_top3(adj_p, xw1, hw_fn, b1_2d, b2_2d):
    grid = (N // TM, NPAD // TK)
    h1, top3 = pl.pallas_call(
        _mm1_body,
        grid=grid,
        in_specs=[pl.BlockSpec((TM, TK), lambda i, k: (i, k)),
                  pl.BlockSpec((TK, D_H), lambda i, k: (k, 0)),
                  pl.BlockSpec((1, D_H), lambda i, k: (0, 0))],
        out_specs=[pl.BlockSpec((TM, D_H), lambda i, k: (i, 0)),
                   pl.BlockSpec((TM, 8), lambda i, k: (i, 0))],
        out_shape=[jax.ShapeDtypeStruct((N, D_H), jnp.float32),
                   jax.ShapeDtypeStruct((N, 8), jnp.int32)],
        scratch_shapes=[pltpu.VMEM((TM, D_H), jnp.float32),
                        pltpu.VMEM((TM, 8), jnp.float32),
                        pltpu.VMEM((TM, 8), jnp.int32)],
        compiler_params=pltpu.CompilerParams(
            dimension_semantics=("arbitrary", "arbitrary")),
    )(adj_p, xw1, b1_2d)

    hw = hw_fn(h1)
    hw_p = jnp.pad(hw, ((0, NPAD - N), (0, 0)))
    emb = pl.pallas_call(
        _mm2_body,
        grid=grid,
        in_specs=[pl.BlockSpec((TM, TK), lambda i, k: (i, k)),
                  pl.BlockSpec((TK, D_OUT), lambda i, k: (k, 0)),
                  pl.BlockSpec((1, D_OUT), lambda i, k: (0, 0))],
        out_specs=pl.BlockSpec((TM, D_OUT), lambda i, k: (i, 0)),
        out_shape=jax.ShapeDtypeStruct((N, D_OUT), jnp.float32),
        scratch_shapes=[pltpu.VMEM((TM, D_OUT), jnp.float32)],
        compiler_params=pltpu.CompilerParams(
            dimension_semantics=("arbitrary", "arbitrary")),
    )(adj_p, hw_p, b2_2d)
    return emb, top3


def _sc_gather(emb, t30, t31, t32, u, v):
    info = plsc.get_sparse_core_info()
    nc, ns = info.num_cores, info.num_subcores
    nw = nc * ns
    bpw = B // nw
    mesh = plsc.VectorSubcoreMesh(core_axis_name="c", subcore_axis_name="s")

    @functools.partial(
        pl.kernel, mesh=mesh,
        out_type=jax.ShapeDtypeStruct((8, B, D_OUT), jnp.float32),
        scratch_types=[
            pltpu.VMEM((bpw,), jnp.int32),
            pltpu.VMEM((bpw,), jnp.int32),
            pltpu.VMEM((bpw,), jnp.int32),
            pltpu.VMEM((bpw,), jnp.int32),
            pltpu.VMEM((bpw,), jnp.int32),
            pltpu.VMEM((bpw,), jnp.int32),
            pltpu.VMEM((bpw,), jnp.int32),
            pltpu.VMEM((bpw,), jnp.int32),
            pltpu.VMEM((bpw, D_OUT), jnp.float32),
            pltpu.SemaphoreType.DMA,
        ],
    )
    def k(emb_hbm, t30_hbm, t31_hbm, t32_hbm, u_hbm, v_hbm, out_hbm,
          u_v, v_v, a0, a1, a2, b0, b1_, b2_, rows_v, sem):
        wid = lax.axis_index("s") * nc + lax.axis_index("c")
        base = wid * bpw
        pltpu.sync_copy(u_hbm.at[pl.ds(base, bpw)], u_v)
        pltpu.sync_copy(v_hbm.at[pl.ds(base, bpw)], v_v)
        pltpu.async_copy(t30_hbm.at[u_v], a0, sem).wait()
        pltpu.async_copy(t31_hbm.at[u_v], a1, sem).wait()
        pltpu.async_copy(t32_hbm.at[u_v], a2, sem).wait()
        pltpu.async_copy(t30_hbm.at[a0], b0, sem).wait()
        pltpu.async_copy(t31_hbm.at[a1], b1_, sem).wait()
        pltpu.async_copy(t32_hbm.at[a2], b2_, sem).wait()
        for j, idxv in enumerate([u_v, v_v, a0, a1, a2, b0, b1_, b2_]):
            pltpu.async_copy(emb_hbm.at[idxv], rows_v, sem).wait()
            pltpu.sync_copy(rows_v, out_hbm.at[j, pl.ds(base, bpw)])

    return k(emb, t30, t31, t32, u, v)


def _tail_body(rows_ref, wq_ref, wk_ref, fcw_ref, fcb_ref, out_ref):
    emb_u = rows_ref[0]
    emb_v = rows_ref[1]
    uv = emb_u + emb_v
    pooled = [(uv + rows_ref[2 + p] + rows_ref[5 + p]) * 0.25
              for p in range(P)]
    q_u = jnp.dot(emb_u, wq_ref[...], preferred_element_type=jnp.float32)
    q_v = jnp.dot(emb_v, wq_ref[...], preferred_element_type=jnp.float32)
    ks = [jnp.dot(pooled[p], wk_ref[...], preferred_element_type=jnp.float32)
          for p in range(P)]
    scale = 1.0 / jnp.sqrt(jnp.float32(D_OUT))

    def attend(q):
        s = [jnp.sum(q * ks[p], axis=1, keepdims=True) * scale
             for p in range(P)]
        m = jnp.maximum(jnp.maximum(s[0], s[1]), s[2])
        e = [jnp.exp(s[p] - m) for p in range(P)]
        den = e[0] + e[1] + e[2]
        ctx = sum(e[p] * pooled[p] for p in range(P)) / den
        return ctx

    hu = emb_u + attend(q_u)
    hv = emb_v + attend(q_v)
    wu = fcw_ref[0, :D_OUT][None, :]
    wv = fcw_ref[0, D_OUT:][None, :]
    res = (jnp.sum(hu * wu, axis=1) + jnp.sum(hv * wv, axis=1)
           + fcb_ref[0, 0])
    out_ref[0, :] = res


def _tail(rows8, wq, wk, fcw_2d, fcb_2d):
    return pl.pallas_call(
        _tail_body,
        out_shape=jax.ShapeDtypeStruct((1, B), jnp.float32),
    )(rows8, wq, wk, fcw_2d, fcb_2d)


def kernel(x, u, v, adj, W1, b1, W2, b2, Wq, Wk, fc_W, fc_b):
    adj_p = jnp.pad(adj, ((0, 0), (0, NPAD - N)))
    x_p = jnp.pad(x, ((0, NPAD - N), (0, 0)))
    xw1 = _small_matmul(x_p, W1, 512)
    emb, top3 = _gcn_top3(
        adj_p, xw1,
        lambda h1: _small_matmul(h1, W2, TM),
        b1.reshape(1, -1), b2.reshape(1, -1))
    t30 = top3[:, 0]
    t31 = top3[:, 1]
    t32 = top3[:, 2]
    rows8 = _sc_gather(emb, t30, t31, t32,
                       u.astype(jnp.int32), v.astype(jnp.int32))
    out = _tail(rows8, Wq, Wk, fc_W.reshape(1, -1), fc_b.reshape(1, 1))
    return out[0]


# speedup vs baseline: 186.7849x; 1.0614x over previous
import functools

import jax
import jax.numpy as jnp
from jax import lax
from jax.experimental import pallas as pl
from jax.experimental.pallas import tpu as pltpu
from jax.experimental.pallas import tpu_sc as plsc

N = 10000
D_IN = 128
D_H = 256
D_OUT = 128
P = 3
L = 4
B = 1024

NPAD = 10240
TM = 200
TK = 2048
BIGI = 2 ** 30


def _xw_body(x_ref, w_ref, o_ref):
    o_ref[...] = jnp.dot(x_ref[...], w_ref[...],
                         preferred_element_type=jnp.float32)


def _small_matmul(x, w, tm):
    m, k = x.shape
    n = w.shape[1]
    return pl.pallas_call(
        _xw_body,
        grid=(m // tm,),
        in_specs=[pl.BlockSpec((tm, k), lambda i: (i, 0)),
                  pl.BlockSpec((k, n), lambda i: (0, 0))],
        out_specs=pl.BlockSpec((tm, n), lambda i: (i, 0)),
        out_shape=jax.ShapeDtypeStruct((m, n), jnp.float32),
    )(x, w)


def _top3_tile(a, iota):
    vals, idxs = [], []
    for _ in range(3):
        m = jnp.max(a, axis=1, keepdims=True)
        ii = jnp.min(jnp.where(a == m, iota, BIGI), axis=1, keepdims=True)
        vals.append(m)
        idxs.append(ii)
        a = jnp.where(iota == ii, -1.0, a)
    return vals, idxs


def _merge3(cv, ci):
    nv, ni = [], []
    for _ in range(3):
        m = jnp.max(cv, axis=1, keepdims=True)
        ii = jnp.min(jnp.where(cv == m, ci, BIGI), axis=1, keepdims=True)
        nv.append(m)
        ni.append(ii)
        cv = jnp.where(ci == ii, -1.0, cv)
    return nv, ni


def _mm1_body(adj_ref, xw1_ref, b1_ref, h1_ref, top3_ref):
    a = adj_ref[...]
    acc = jnp.dot(a, xw1_ref[...], preferred_element_type=jnp.float32)
    h1_ref[...] = jnp.maximum(acc + b1_ref[...], 0.0)
    tv, ti = None, None
    for w in range(0, NPAD, TK):
        iota = lax.broadcasted_iota(jnp.int32, (TM, TK), 1) + w
        vals, idxs = _top3_tile(a[:, w:w + TK], iota)
        if tv is None:
            tv, ti = vals, idxs
        else:
            cv = jnp.concatenate(tv + vals, axis=1)
            ci = jnp.concatenate(ti + idxs, axis=1)
            tv, ti = _merge3(cv, ci)
    pad = jnp.full((TM, 5), BIGI, jnp.int32)
    top3_ref[...] = jnp.concatenate(ti + [pad], axis=1)


def _mm2_body(adj_ref, hw_ref, b2_ref, emb_ref):
    emb_ref[...] = jnp.dot(adj_ref[...], hw_ref[...],
                           preferred_element_type=jnp.float32) + b2_ref[...]


def _gcn_top3(adj_p, xw1, hw_fn, b1_2d, b2_2d):
    grid = (N // TM,)
    h1, top3 = pl.pallas_call(
        _mm1_body,
        grid=grid,
        in_specs=[pl.BlockSpec((TM, NPAD), lambda i: (i, 0)),
                  pl.BlockSpec((NPAD, D_H), lambda i: (0, 0)),
                  pl.BlockSpec((1, D_H), lambda i: (0, 0))],
        out_specs=[pl.BlockSpec((TM, D_H), lambda i: (i, 0)),
                   pl.BlockSpec((TM, 8), lambda i: (i, 0))],
        out_shape=[jax.ShapeDtypeStruct((N, D_H), jnp.float32),
                   jax.ShapeDtypeStruct((N, 8), jnp.int32)],
        compiler_params=pltpu.CompilerParams(
            dimension_semantics=("arbitrary",)),
    )(adj_p, xw1, b1_2d)

    hw = hw_fn(h1)
    hw_p = jnp.pad(hw, ((0, NPAD - N), (0, 0)))
    emb = pl.pallas_call(
        _mm2_body,
        grid=grid,
        in_specs=[pl.BlockSpec((TM, NPAD), lambda i: (i, 0)),
                  pl.BlockSpec((NPAD, D_OUT), lambda i: (0, 0)),
                  pl.BlockSpec((1, D_OUT), lambda i: (0, 0))],
        out_specs=pl.BlockSpec((TM, D_OUT), lambda i: (i, 0)),
        out_shape=jax.ShapeDtypeStruct((N, D_OUT), jnp.float32),
        compiler_params=pltpu.CompilerParams(
            dimension_semantics=("arbitrary",)),
    )(adj_p, hw_p, b2_2d)
    return emb, top3


def _sc_gather(emb, t30, t31, t32, u, v):
    info = plsc.get_sparse_core_info()
    nc, ns = info.num_cores, info.num_subcores
    nw = nc * ns
    bpw = B // nw
    mesh = plsc.VectorSubcoreMesh(core_axis_name="c", subcore_axis_name="s")

    @functools.partial(
        pl.kernel, mesh=mesh,
        out_type=jax.ShapeDtypeStruct((8, B, D_OUT), jnp.float32),
        scratch_types=[
            pltpu.VMEM((bpw,), jnp.int32),
            pltpu.VMEM((bpw,), jnp.int32),
            pltpu.VMEM((bpw,), jnp.int32),
            pltpu.VMEM((bpw,), jnp.int32),
            pltpu.VMEM((bpw,), jnp.int32),
            pltpu.VMEM((bpw,), jnp.int32),
            pltpu.VMEM((bpw,), jnp.int32),
            pltpu.VMEM((bpw,), jnp.int32),
            pltpu.VMEM((bpw, D_OUT), jnp.float32),
            pltpu.SemaphoreType.DMA,
        ],
    )
    def k(emb_hbm, t30_hbm, t31_hbm, t32_hbm, u_hbm, v_hbm, out_hbm,
          u_v, v_v, a0, a1, a2, b0, b1_, b2_, rows_v, sem):
        wid = lax.axis_index("s") * nc + lax.axis_index("c")
        base = wid * bpw
        pltpu.sync_copy(u_hbm.at[pl.ds(base, bpw)], u_v)
        pltpu.sync_copy(v_hbm.at[pl.ds(base, bpw)], v_v)
        pltpu.async_copy(t30_hbm.at[u_v], a0, sem).wait()
        pltpu.async_copy(t31_hbm.at[u_v], a1, sem).wait()
        pltpu.async_copy(t32_hbm.at[u_v], a2, sem).wait()
        pltpu.async_copy(t30_hbm.at[a0], b0, sem).wait()
        pltpu.async_copy(t31_hbm.at[a1], b1_, sem).wait()
        pltpu.async_copy(t32_hbm.at[a2], b2_, sem).wait()
        for j, idxv in enumerate([u_v, v_v, a0, a1, a2, b0, b1_, b2_]):
            pltpu.async_copy(emb_hbm.at[idxv], rows_v, sem).wait()
            pltpu.sync_copy(rows_v, out_hbm.at[j, pl.ds(base, bpw)])

    return k(emb, t30, t31, t32, u, v)


def _tail_body(rows_ref, wq_ref, wk_ref, fcw_ref, fcb_ref, out_ref):
    emb_u = rows_ref[0]
    emb_v = rows_ref[1]
    uv = emb_u + emb_v
    pooled = [(uv + rows_ref[2 + p] + rows_ref[5 + p]) * 0.25
              for p in range(P)]
    q_u = jnp.dot(emb_u, wq_ref[...], preferred_element_type=jnp.float32)
    q_v = jnp.dot(emb_v, wq_ref[...], preferred_element_type=jnp.float32)
    ks = [jnp.dot(pooled[p], wk_ref[...], preferred_element_type=jnp.float32)
          for p in range(P)]
    scale = 1.0 / jnp.sqrt(jnp.float32(D_OUT))

    def attend(q):
        s = [jnp.sum(q * ks[p], axis=1, keepdims=True) * scale
             for p in range(P)]
        m = jnp.maximum(jnp.maximum(s[0], s[1]), s[2])
        e = [jnp.exp(s[p] - m) for p in range(P)]
        den = e[0] + e[1] + e[2]
        ctx = sum(e[p] * pooled[p] for p in range(P)) / den
        return ctx

    hu = emb_u + attend(q_u)
    hv = emb_v + attend(q_v)
    wu = fcw_ref[0, :D_OUT][None, :]
    wv = fcw_ref[0, D_OUT:][None, :]
    res = (jnp.sum(hu * wu, axis=1) + jnp.sum(hv * wv, axis=1)
           + fcb_ref[0, 0])
    out_ref[0, :] = res


def _tail(rows8, wq, wk, fcw_2d, fcb_2d):
    return pl.pallas_call(
        _tail_body,
        out_shape=jax.ShapeDtypeStruct((1, B), jnp.float32),
    )(rows8, wq, wk, fcw_2d, fcb_2d)


def kernel(x, u, v, adj, W1, b1, W2, b2, Wq, Wk, fc_W, fc_b):
    adj_p = jnp.pad(adj, ((0, 0), (0, NPAD - N)))
    x_p = jnp.pad(x, ((0, NPAD - N), (0, 0)))
    xw1 = _small_matmul(x_p, W1, 512)
    emb, top3 = _gcn_top3(
        adj_p, xw1,
        lambda h1: _small_matmul(h1, W2, TM),
        b1.reshape(1, -1), b2.reshape(1, -1))
    t30 = top3[:, 0]
    t31 = top3[:, 1]
    t32 = top3[:, 2]
    rows8 = _sc_gather(emb, t30, t31, t32,
                       u.astype(jnp.int32), v.astype(jnp.int32))
    out = _tail(rows8, Wq, Wk, fc_W.reshape(1, -1), fc_b.reshape(1, 1))
    return out[0]
